# baseline (speedup 1.0000x reference)
import jax
import jax.numpy as jnp
from jax import lax
from jax.experimental import pallas as pl
from jax.experimental.pallas import tpu as pltpu
from jax.experimental.pallas import tpu_sc as plsc

VOCAB = 100000
HIDDEN = 16
BATCH = 1024


_NUM_CORES = 2
_NUM_SUBCORES = 16
_NUM_WORKERS = _NUM_CORES * _NUM_SUBCORES
_B_PER_W = BATCH // _NUM_WORKERS


def _gather_body(table_hbm, idx_hbm, out_hbm, idx_v, rows_v, sem):
    wid = lax.axis_index("s") * _NUM_CORES + lax.axis_index("c")
    base = wid * _B_PER_W
    pltpu.sync_copy(idx_hbm.at[pl.ds(base, _B_PER_W)], idx_v)
    pltpu.async_copy(table_hbm.at[idx_v], rows_v, sem).wait()
    pltpu.sync_copy(rows_v, out_hbm.at[pl.ds(base, _B_PER_W)])


def _sc_gather(table, idx):
    mesh = plsc.VectorSubcoreMesh(core_axis_name="c", subcore_axis_name="s")
    return pl.kernel(
        _gather_body,
        out_type=jax.ShapeDtypeStruct((BATCH, HIDDEN), jnp.float32),
        mesh=mesh,
        scratch_types=[
            pltpu.VMEM((_B_PER_W,), jnp.int32),
            pltpu.VMEM((_B_PER_W, HIDDEN), jnp.float32),
            pltpu.SemaphoreType.DMA,
        ],
        compiler_params=pltpu.CompilerParams(use_tc_tiling_on_sc=False),
    )(table, idx)


_VT = 2048
_NB = pl.cdiv(VOCAB, _VT)
_W_LAST = VOCAB - (_NB - 1) * _VT
_NBUF = 4


def _proj_body(h_ref, w_ref, b_ref, o_hbm, buf, tail_buf, sem, tail_sem):
    j = pl.program_id(0)
    slot = lax.rem(j, _NBUF)

    acc = lax.dot_general(
        h_ref[...], w_ref[...],
        (((1,), (1,)), ((), ())),
        preferred_element_type=jnp.float32,
    ) + b_ref[...]

    @pl.when(j < _NB - 1)
    def _():
        for k in range(_NBUF):
            @pl.when(slot == k)
            def _(k=k):
                @pl.when(j >= _NBUF)
                def _():
                    pltpu.make_async_copy(
                        buf.at[k],
                        o_hbm.at[:, pl.ds((j - _NBUF) * _VT, _VT)],
                        sem.at[k],
                    ).wait()

                buf[k] = acc
                pltpu.make_async_copy(
                    buf.at[k],
                    o_hbm.at[:, pl.ds(j * _VT, _VT)],
                    sem.at[k],
                ).start()

    @pl.when(j == _NB - 1)
    def _():
        tail_buf[...] = acc[:, :_W_LAST]
        pltpu.make_async_copy(
            tail_buf,
            o_hbm.at[:, pl.ds((_NB - 1) * _VT, _W_LAST)],
            tail_sem,
        ).start()
        for jj in range(max(_NB - 1 - _NBUF, 0), _NB - 1):
            s = jj % _NBUF
            pltpu.make_async_copy(
                buf.at[s],
                o_hbm.at[:, pl.ds(jj * _VT, _VT)],
                sem.at[s],
            ).wait()
        pltpu.make_async_copy(
            tail_buf,
            o_hbm.at[:, pl.ds((_NB - 1) * _VT, _W_LAST)],
            tail_sem,
        ).wait()


def _tc_project(hidden, proj_weight, bias2d):
    return pl.pallas_call(
        _proj_body,
        grid=(_NB,),
        in_specs=[
            pl.BlockSpec((BATCH, HIDDEN), lambda j: (0, 0)),
            pl.BlockSpec((_VT, HIDDEN), lambda j: (j, 0)),
            pl.BlockSpec((1, _VT), lambda j: (0, j)),
        ],
        out_specs=pl.BlockSpec(memory_space=pl.ANY),
        out_shape=jax.ShapeDtypeStruct((BATCH, VOCAB), jnp.float32),
        scratch_shapes=[
            pltpu.VMEM((_NBUF, BATCH, _VT), jnp.float32),
            pltpu.VMEM((BATCH, _W_LAST), jnp.float32),
            pltpu.SemaphoreType.DMA((_NBUF,)),
            pltpu.SemaphoreType.DMA,
        ],
        compiler_params=pltpu.CompilerParams(
            dimension_semantics=("arbitrary",),
        ),
    )(hidden, proj_weight, bias2d)


def kernel(input_ids, embed_weight, proj_weight, proj_bias):
    hidden = _sc_gather(embed_weight, input_ids.astype(jnp.int32))
    bias2d = proj_bias.reshape(1, VOCAB)
    return _tc_project(hidden.astype(jnp.bfloat16),
                       proj_weight.astype(jnp.bfloat16), bias2d)
